# baseline (speedup 1.0000x reference)
import functools

import jax
import jax.numpy as jnp
from jax import lax
from jax.experimental import pallas as pl
from jax.experimental.pallas import tpu as pltpu
from jax.experimental.pallas import tpu_sc as plsc

AVG = 15
MAXW = 3
GRP = AVG * MAXW
LANES = 16
CHUNK = 120
PTILES = 12


def _encoder_body(nq, n_chunks, out_per_w, subs_per_w, scale_w, scale_p,
                  src_ref, w_ref, p_ref, out_ref,
                  idx2, rows, pbuf, stage, pp, out_v, shared,
                  sem_a, sem_b):
    nc = lax.axis_index("c")
    ns = lax.axis_index("s")
    wid = ns * 2 + nc
    toks_per_w = subs_per_w * AVG
    tok0 = wid * toks_per_w
    half = n_chunks // 2 + 1

    copies = []
    for k in range(n_chunks):
        pltpu.sync_copy(src_ref.at[pl.ds(tok0 + k * CHUNK, CHUNK)],
                        idx2.at[k])
        sem = sem_a if k < half else sem_b
        copies.append(pltpu.async_copy(
            w_ref.at[idx2.at[k]], rows.at[pl.ds(k * CHUNK, CHUNK)], sem))

    g_per_t = subs_per_w // PTILES
    rows_per_t = g_per_t * AVG

    @pl.when(ns < PTILES)
    def _pool():
        p0 = nc * toks_per_w + ns * rows_per_t
        pltpu.sync_copy(p_ref.at[pl.ds(p0, rows_per_t)], pbuf)
        for t in range(g_per_t):
            base = t * AVG
            for q in range(nq):
                sl = pl.ds(q * LANES, LANES)
                acc = pbuf[base, sl]
                for i in range(1, AVG):
                    acc = acc + pbuf[base + i, sl]
                stage[t, sl] = acc * scale_p
        pltpu.sync_copy(stage, shared.at[pl.ds(ns * g_per_t, g_per_t)])

    plsc.subcore_barrier()
    pltpu.sync_copy(shared, pp)

    def compute(j, _):
        r0 = j * GRP
        res = None
        for kk in range(MAXW):
            b0 = r0 + kk * AVG
            accs = [None] * nq
            for i in range(AVG):
                for h in range(nq // 2):
                    v = rows[b0 + i, pl.ds(h * 2 * LANES, 2 * LANES)]
                    a, b = plsc.unpack(v, format=plsc.PackFormat.INTERLEAVED)
                    if accs[2 * h] is None:
                        accs[2 * h], accs[2 * h + 1] = a, b
                    else:
                        accs[2 * h] = accs[2 * h] + a
                        accs[2 * h + 1] = accs[2 * h + 1] + b
            es = [accs[q] * scale_w + pp[j * MAXW + kk, pl.ds(q * LANES, LANES)]
                  for q in range(nq)]
            if res is None:
                res = es
            else:
                res = [jnp.maximum(a, b) for a, b in zip(res, es)]
        for q in range(nq):
            out_v[j, pl.ds(q * LANES, LANES)] = res[q]
        return 0

    for k in range(half):
        copies[k].wait()
    j_mid = (half * CHUNK) // GRP
    lax.fori_loop(0, j_mid, compute, 0)

    for k in range(half, n_chunks):
        copies[k].wait()
    lax.fori_loop(j_mid, out_per_w, compute, 0)

    pltpu.sync_copy(out_v, out_ref.at[pl.ds(wid * out_per_w, out_per_w)])


@functools.partial(jax.jit, static_argnums=(3, 4, 5))
def _encode(src_flat, w, p, n_out, d, n_workers):
    out_per_w = n_out // n_workers
    subs_per_w = out_per_w * MAXW
    n_chunks = (subs_per_w * AVG) // CHUNK
    nq = d // LANES
    scale_w = float(d) ** 0.5 / AVG
    scale_p = 1.0 / AVG
    mesh = plsc.VectorSubcoreMesh(core_axis_name="c", subcore_axis_name="s")
    body = functools.partial(_encoder_body, nq, n_chunks, out_per_w,
                             subs_per_w, scale_w, scale_p)
    return pl.kernel(
        body,
        out_type=jax.ShapeDtypeStruct((n_out, d), jnp.float32),
        mesh=mesh,
        compiler_params=pltpu.CompilerParams(use_tc_tiling_on_sc=False,
                                             needs_layout_passes=False),
        scratch_types=[
            pltpu.VMEM((n_chunks, CHUNK), jnp.int32),
            pltpu.VMEM((n_chunks * CHUNK, d), jnp.bfloat16),
            pltpu.VMEM((subs_per_w // PTILES * AVG, d), jnp.float32),
            pltpu.VMEM((subs_per_w // PTILES, d), jnp.float32),
            pltpu.VMEM((subs_per_w, d), jnp.float32),
            pltpu.VMEM((out_per_w, d), jnp.float32),
            pltpu.VMEM_SHARED((subs_per_w, d), jnp.float32),
            pltpu.SemaphoreType.DMA,
            pltpu.SemaphoreType.DMA,
        ],
    )(src_flat, w, p)


def kernel(src, W, P):
    b, l = src.shape
    v, d = W.shape
    n_out = b * (l // GRP)
    w16 = (W.astype(jnp.bfloat16)
            .reshape(v, d // 32, 2, LANES)
            .swapaxes(2, 3)
            .reshape(v, d))
    out = _encode(src.reshape(b * l), w16, P, n_out, d, 32)
    return out.reshape(b, l // GRP, d)


# speedup vs baseline: 1.4643x; 1.4643x over previous
import functools

import jax
import jax.numpy as jnp
from jax import lax
from jax.experimental import pallas as pl
from jax.experimental.pallas import tpu as pltpu
from jax.experimental.pallas import tpu_sc as plsc

AVG = 15
MAXW = 3
GRP = AVG * MAXW
LANES = 16
CHUNK = 120
HMASK = -65536


def _encoder_body(nq, n_chunks, out_per_w, subs_per_w, scale_w, scale_p,
                  src_ref, w_ref, p_ref, out_ref,
                  idx2, rows, pbuf, stage, pp, out_v, shared,
                  sem_a, sem_b):
    ns = lax.axis_index("s")
    toks_per_w = subs_per_w * AVG
    tok0 = ns * toks_per_w
    half = n_chunks // 2

    copies = []
    for k in range(n_chunks):
        pltpu.sync_copy(src_ref.at[pl.ds(tok0 + k * CHUNK, CHUNK)],
                        idx2.at[k])
        sem = sem_a if k < half else sem_b
        copies.append(pltpu.async_copy(
            w_ref.at[idx2.at[k]], rows.at[pl.ds(k * CHUNK, CHUNK)], sem))

    g_per_t = subs_per_w // 16
    rows_per_t = g_per_t * AVG
    p0 = ns * rows_per_t
    pltpu.sync_copy(p_ref.at[pl.ds(p0, rows_per_t)], pbuf)
    for t in range(g_per_t):
        base = t * AVG
        for q in range(nq):
            sl = pl.ds(q * LANES, LANES)
            acc = pbuf[base, sl]
            for i in range(1, AVG):
                acc = acc + pbuf[base + i, sl]
            stage[t, sl] = acc * scale_p
    pltpu.sync_copy(stage, shared.at[pl.ds(ns * g_per_t, g_per_t)])
    plsc.subcore_barrier()
    pltpu.sync_copy(shared, pp)

    def compute(j, _):
        r0 = j * GRP
        res = None
        for kk in range(MAXW):
            b0 = r0 + kk * AVG
            accs = [None] * nq
            for i in range(AVG):
                for h in range(nq // 2):
                    w = rows[b0 + i, pl.ds(h * LANES, LANES)]
                    a = plsc.bitcast(w << 16, jnp.float32)
                    b = plsc.bitcast(w & HMASK, jnp.float32)
                    if accs[2 * h] is None:
                        accs[2 * h], accs[2 * h + 1] = a, b
                    else:
                        accs[2 * h] = accs[2 * h] + a
                        accs[2 * h + 1] = accs[2 * h + 1] + b
            es = [accs[q] * scale_w + pp[j * MAXW + kk, pl.ds(q * LANES, LANES)]
                  for q in range(nq)]
            if res is None:
                res = es
            else:
                res = [jnp.maximum(x, y) for x, y in zip(res, es)]
        for q in range(nq):
            out_v[j, pl.ds(q * LANES, LANES)] = res[q]
        return 0

    for k in range(half):
        copies[k].wait()
    j_mid = (half * CHUNK) // GRP
    lax.fori_loop(0, j_mid, compute, 0)

    for k in range(half, n_chunks):
        copies[k].wait()
    lax.fori_loop(j_mid, out_per_w, compute, 0)

    pltpu.sync_copy(out_v, out_ref.at[pl.ds(ns * out_per_w, out_per_w)])


@functools.partial(jax.jit, static_argnums=(3, 4, 5))
def _encode(src_flat, w_packed, p, n_out, d, n_workers):
    out_per_w = n_out // n_workers
    subs_per_w = out_per_w * MAXW
    n_chunks = (subs_per_w * AVG) // CHUNK
    nq = d // LANES
    scale_w = float(d) ** 0.5 / AVG
    scale_p = 1.0 / AVG
    mesh = plsc.VectorSubcoreMesh(core_axis_name="c", subcore_axis_name="s",
                                  num_cores=1)
    body = functools.partial(_encoder_body, nq, n_chunks, out_per_w,
                             subs_per_w, scale_w, scale_p)
    return pl.kernel(
        body,
        out_type=jax.ShapeDtypeStruct((n_out, d), jnp.float32),
        mesh=mesh,
        compiler_params=pltpu.CompilerParams(use_tc_tiling_on_sc=False,
                                             needs_layout_passes=False),
        scratch_types=[
            pltpu.VMEM((n_chunks, CHUNK), jnp.int32),
            pltpu.VMEM((n_chunks * CHUNK, d // 2), jnp.int32),
            pltpu.VMEM((subs_per_w // 16 * AVG, d), jnp.float32),
            pltpu.VMEM((subs_per_w // 16, d), jnp.float32),
            pltpu.VMEM((subs_per_w, d), jnp.float32),
            pltpu.VMEM((out_per_w, d), jnp.float32),
            pltpu.VMEM_SHARED((subs_per_w, d), jnp.float32),
            pltpu.SemaphoreType.DMA,
            pltpu.SemaphoreType.DMA,
        ],
    )(src_flat, w_packed, p)


def kernel(src, W, P):
    b, l = src.shape
    v, d = W.shape
    n_out = b * (l // GRP)
    w16 = (W.astype(jnp.bfloat16)
            .reshape(v, d // 32, 2, LANES)
            .swapaxes(2, 3))
    w_packed = lax.bitcast_convert_type(w16, jnp.int32).reshape(v, d // 2)
    out = _encode(src.reshape(b * l), w_packed, P, n_out, d, 16)
    return out.reshape(b, l // GRP, d)
